# baseline (speedup 1.0000x reference)
import functools

import jax
import jax.numpy as jnp
from jax import lax
from jax.experimental import pallas as pl
from jax.experimental.pallas import tpu as pltpu
from jax.experimental.pallas import tpu_sc as plsc

B, N, D = 64, 576, 768
K = 288
BB = 8
SPLITS = (24, 40)

NW = 32


def _chunks_for(rows_per_w):
    out = []
    left = rows_per_w
    while left > 64:
        out.append(64)
        left -= 64
    out.append(left)
    return tuple(out)


def _total_order_key(v):
    bits = lax.bitcast_convert_type(v, jnp.int32)
    return bits ^ ((bits >> 31) & jnp.int32(0x7FFFFFFF))


def _make_rank_body(row_off):
    def _rank_body(noise_ref, noise_t_ref, idx_ref):
        kb = _total_order_key(noise_ref[...])
        kt = _total_order_key(noise_t_ref[0])
        ii = lax.broadcasted_iota(jnp.int32, (N, 1), 0)
        jj = lax.broadcasted_iota(jnp.int32, (1, N), 1)
        rr = lax.broadcasted_iota(jnp.int32, (1, K), 1)
        pieces = []
        for b in range(BB):
            ki = kt[:, b:b + 1]
            kj = kb[b:b + 1, :]
            beats = (kj > ki) | ((kj == ki) & (jj < ii))
            rank = jnp.sum(beats.astype(jnp.int32), axis=1, keepdims=True)
            sel = (rank == rr).astype(jnp.int32)
            idxv = jnp.sum(sel * ii, axis=0, keepdims=True)
            row = row_off + pl.program_id(0) * BB + b
            pieces.append(idxv + row * N)
        idx_ref[0, 0, :] = jnp.concatenate(pieces, axis=1).reshape(BB * K)
    return _rank_body


def _topk_flat_indices(noise, noise_t_all, row_off, bh):
    g0 = row_off // BB
    return pl.pallas_call(
        _make_rank_body(row_off),
        grid=(bh // BB,),
        in_specs=[
            pl.BlockSpec((BB, N), lambda i: (g0 + i, 0)),
            pl.BlockSpec((1, N, BB), lambda i: (g0 + i, 0, 0)),
        ],
        out_specs=pl.BlockSpec((1, 1, BB * K), lambda i: (i, 0, 0)),
        out_shape=jax.ShapeDtypeStruct((bh // BB, 1, BB * K), jnp.int32),
    )(noise, noise_t_all)


def _scratch_types(rows_per_w):
    return [
        pltpu.VMEM((rows_per_w,), jnp.int32),
        pltpu.VMEM((64, D), jnp.float32),
        pltpu.VMEM((64, D), jnp.float32),
        pltpu.SemaphoreType.DMA,
        pltpu.SemaphoreType.DMA,
    ]


def _gather_worker(x_hbm, idx_hbm, out_hbm, idx_v, buf0, buf1, sem0, sem1,
                   out_off, rows_per_w):
    chunks = _chunks_for(rows_per_w)
    wid = lax.axis_index("s") * 2 + lax.axis_index("c")
    base = wid * rows_per_w
    pltpu.sync_copy(idx_hbm.at[pl.ds(base, rows_per_w)], idx_v)
    bufs = (buf0, buf1)
    sems = (sem0, sem1)
    offs = [0]
    for c in chunks:
        offs.append(offs[-1] + c)
    nch = len(chunks)
    copies = [None] * nch
    copies[0] = pltpu.async_copy(
        x_hbm.at[idx_v.at[pl.ds(0, chunks[0])]],
        bufs[0].at[pl.ds(0, chunks[0])], sems[0])
    for c in range(nch):
        if c + 1 < nch:
            copies[c + 1] = pltpu.async_copy(
                x_hbm.at[idx_v.at[pl.ds(offs[c + 1], chunks[c + 1])]],
                bufs[(c + 1) % 2].at[pl.ds(0, chunks[c + 1])],
                sems[(c + 1) % 2])
        copies[c].wait()
        pltpu.sync_copy(
            bufs[c % 2].at[pl.ds(0, chunks[c])],
            out_hbm.at[pl.ds(out_off + base + offs[c], chunks[c])])


def _sc_gather_first(x_flat, idx_flat, bh):
    rows_per_w = (bh * K) // NW
    mesh = plsc.VectorSubcoreMesh(core_axis_name="c", subcore_axis_name="s")

    @functools.partial(
        pl.kernel, mesh=mesh,
        out_type=jax.ShapeDtypeStruct((B * K, D), jnp.float32),
        scratch_types=_scratch_types(rows_per_w),
    )
    def gather_kernel(x_hbm, idx_hbm, out_hbm, *scr):
        _gather_worker(x_hbm, idx_hbm, out_hbm, *scr,
                       out_off=0, rows_per_w=rows_per_w)

    return gather_kernel(x_flat, idx_flat)


def _sc_gather_into(x_flat, idx_flat, out_ref, out_off, bh):
    rows_per_w = (bh * K) // NW
    mesh = plsc.VectorSubcoreMesh(core_axis_name="c", subcore_axis_name="s")

    @functools.partial(
        pl.kernel, mesh=mesh,
        scratch_types=_scratch_types(rows_per_w),
    )
    def gather_kernel(x_hbm, idx_hbm, out_hbm, *scr):
        _gather_worker(x_hbm, idx_hbm, out_hbm, *scr,
                       out_off=out_off, rows_per_w=rows_per_w)

    gather_kernel(x_flat, idx_flat, out_ref)


@jax.jit
def kernel(x, noise):
    x_flat = x.reshape(B * N, D)
    noise_t_all = noise.reshape(B // BB, BB, N).transpose(0, 2, 1)
    row0 = 0
    idxs = []
    for bh in SPLITS:
        idxs.append(_topk_flat_indices(noise, noise_t_all, row0, bh))
        row0 += bh
    out = _sc_gather_first(x_flat, idxs[0].reshape(SPLITS[0] * K), SPLITS[0])
    out_ref = jax.new_ref(out)
    row0 = SPLITS[0]
    for s, bh in enumerate(SPLITS[1:], start=1):
        _sc_gather_into(x_flat, idxs[s].reshape(bh * K), out_ref,
                        row0 * K, bh)
        row0 += bh
    return out_ref[...].reshape(B, K, D)


# speedup vs baseline: 1.0131x; 1.0131x over previous
import functools

import jax
import jax.numpy as jnp
from jax import lax
from jax.experimental import pallas as pl
from jax.experimental.pallas import tpu as pltpu
from jax.experimental.pallas import tpu_sc as plsc

B, N, D = 64, 576, 768
K = 288
BB = 8
SPLITS = (16, 16, 32)

NW = 32


def _chunks_for(rows_per_w):
    out = [16]
    left = rows_per_w - 16
    while left > 64:
        out.append(64)
        left -= 64
    out.append(left)
    return tuple(out)


def _total_order_key(v):
    bits = lax.bitcast_convert_type(v, jnp.int32)
    return bits ^ ((bits >> 31) & jnp.int32(0x7FFFFFFF))


def _make_rank_body(row_off):
    def _rank_body(noise_ref, noise_t_ref, idx_ref):
        kb = _total_order_key(noise_ref[...])
        kt = _total_order_key(noise_t_ref[0])
        ii = lax.broadcasted_iota(jnp.int32, (N, 1), 0)
        jj = lax.broadcasted_iota(jnp.int32, (1, N), 1)
        rr = lax.broadcasted_iota(jnp.int32, (1, K), 1)
        pieces = []
        for b in range(BB):
            ki = kt[:, b:b + 1]
            kj = kb[b:b + 1, :]
            beats = (kj > ki) | ((kj == ki) & (jj < ii))
            rank = jnp.sum(beats.astype(jnp.int32), axis=1, keepdims=True)
            sel = (rank == rr).astype(jnp.int32)
            idxv = jnp.sum(sel * ii, axis=0, keepdims=True)
            row = row_off + pl.program_id(0) * BB + b
            pieces.append(idxv + row * N)
        idx_ref[0, 0, :] = jnp.concatenate(pieces, axis=1).reshape(BB * K)
    return _rank_body


def _topk_flat_indices(noise, noise_t_all, row_off, bh):
    g0 = row_off // BB
    return pl.pallas_call(
        _make_rank_body(row_off),
        grid=(bh // BB,),
        in_specs=[
            pl.BlockSpec((BB, N), lambda i: (g0 + i, 0)),
            pl.BlockSpec((1, N, BB), lambda i: (g0 + i, 0, 0)),
        ],
        out_specs=pl.BlockSpec((1, 1, BB * K), lambda i: (i, 0, 0)),
        out_shape=jax.ShapeDtypeStruct((bh // BB, 1, BB * K), jnp.int32),
    )(noise, noise_t_all)


def _scratch_types(rows_per_w):
    return [
        pltpu.VMEM((rows_per_w,), jnp.int32),
        pltpu.VMEM((64, D), jnp.float32),
        pltpu.VMEM((64, D), jnp.float32),
        pltpu.SemaphoreType.DMA,
        pltpu.SemaphoreType.DMA,
    ]


def _gather_worker(x_hbm, idx_hbm, out_hbm, idx_v, buf0, buf1, sem0, sem1,
                   out_off, rows_per_w):
    chunks = _chunks_for(rows_per_w)
    wid = lax.axis_index("s") * 2 + lax.axis_index("c")
    base = wid * rows_per_w
    pltpu.sync_copy(idx_hbm.at[pl.ds(base, rows_per_w)], idx_v)
    bufs = (buf0, buf1)
    sems = (sem0, sem1)
    offs = [0]
    for c in chunks:
        offs.append(offs[-1] + c)
    nch = len(chunks)
    copies = [None] * nch
    copies[0] = pltpu.async_copy(
        x_hbm.at[idx_v.at[pl.ds(0, chunks[0])]],
        bufs[0].at[pl.ds(0, chunks[0])], sems[0])
    for c in range(nch):
        if c + 1 < nch:
            copies[c + 1] = pltpu.async_copy(
                x_hbm.at[idx_v.at[pl.ds(offs[c + 1], chunks[c + 1])]],
                bufs[(c + 1) % 2].at[pl.ds(0, chunks[c + 1])],
                sems[(c + 1) % 2])
        copies[c].wait()
        pltpu.sync_copy(
            bufs[c % 2].at[pl.ds(0, chunks[c])],
            out_hbm.at[pl.ds(out_off + base + offs[c], chunks[c])])


def _sc_gather_first(x_flat, idx_flat, bh):
    rows_per_w = (bh * K) // NW
    mesh = plsc.VectorSubcoreMesh(core_axis_name="c", subcore_axis_name="s")

    @functools.partial(
        pl.kernel, mesh=mesh,
        out_type=jax.ShapeDtypeStruct((B * K, D), jnp.float32),
        scratch_types=_scratch_types(rows_per_w),
    )
    def gather_kernel(x_hbm, idx_hbm, out_hbm, *scr):
        _gather_worker(x_hbm, idx_hbm, out_hbm, *scr,
                       out_off=0, rows_per_w=rows_per_w)

    return gather_kernel(x_flat, idx_flat)


def _sc_gather_into(x_flat, idx_flat, out_ref, out_off, bh):
    rows_per_w = (bh * K) // NW
    mesh = plsc.VectorSubcoreMesh(core_axis_name="c", subcore_axis_name="s")

    @functools.partial(
        pl.kernel, mesh=mesh,
        scratch_types=_scratch_types(rows_per_w),
    )
    def gather_kernel(x_hbm, idx_hbm, out_hbm, *scr):
        _gather_worker(x_hbm, idx_hbm, out_hbm, *scr,
                       out_off=out_off, rows_per_w=rows_per_w)

    gather_kernel(x_flat, idx_flat, out_ref)


@jax.jit
def kernel(x, noise):
    x_flat = x.reshape(B * N, D)
    noise_t_all = noise.reshape(B // BB, BB, N).transpose(0, 2, 1)
    row0 = 0
    idxs = []
    for bh in SPLITS:
        idxs.append(_topk_flat_indices(noise, noise_t_all, row0, bh))
        row0 += bh
    out = _sc_gather_first(x_flat, idxs[0].reshape(SPLITS[0] * K), SPLITS[0])
    out_ref = jax.new_ref(out)
    row0 = SPLITS[0]
    for s, bh in enumerate(SPLITS[1:], start=1):
        _sc_gather_into(x_flat, idxs[s].reshape(bh * K), out_ref,
                        row0 * K, bh)
        row0 += bh
    return out_ref[...].reshape(B, K, D)
